# baseline (speedup 1.0000x reference)
import functools

import jax
import jax.numpy as jnp
from jax import lax
from jax.experimental import pallas as pl
from jax.experimental.pallas import tpu as pltpu
from jax.experimental.pallas import tpu_sc as plsc

B, S, D = 4, 8192, 768
G, EPG = 4, 16
E = G * EPG
H = 64
T = B * S
WPAD = 192
LROWS = 128
BT = 512

NC, NS = 2, 16
NW = NC * NS
TPW = T // NW
CH = 128
NCHUNK = TPW // CH
NTILE = CH // 16
EP = 72


def _tc_logits_body(xref, wref, biasref, w2ref, b2ref, outref):
    yt = lax.dot_general(wref[...], xref[...], (((1,), (1,)), ((), ())),
                         preferred_element_type=jnp.float32)
    r = jnp.maximum(yt + biasref[...], 0.0)
    cpre = lax.dot_general(w2ref[...], r, (((1,), (0,)), ((), ())),
                           preferred_element_type=jnp.float32)
    c = jax.nn.sigmoid(cpre + b2ref[0, 0])
    panel = jnp.concatenate(
        [yt[:E], yt[128:132], c, jnp.zeros((LROWS - E - G - 1, BT), jnp.float32)],
        axis=0)
    for cch in range(BT // CH):
        outref[cch] = panel[:, cch * CH:(cch + 1) * CH]


def _tc_logits(xf, wall, bias, w2pad, b2s):
    return pl.pallas_call(
        _tc_logits_body,
        grid=(T // BT,),
        in_specs=[
            pl.BlockSpec((BT, D), lambda i: (i, 0)),
            pl.BlockSpec((WPAD, D), lambda i: (0, 0)),
            pl.BlockSpec((WPAD, 1), lambda i: (0, 0)),
            pl.BlockSpec((1, WPAD), lambda i: (0, 0)),
            pl.BlockSpec(memory_space=pltpu.SMEM),
        ],
        out_specs=pl.BlockSpec((BT // CH, LROWS, CH), lambda i: (i, 0, 0)),
        out_shape=jax.ShapeDtypeStruct((T // CH, LROWS, CH), jnp.float32),
    )(xf, wall, bias, w2pad, b2s)


def _sc_route_body(lg_hbm, disp_hbm, comb_hbm, rp_hbm, inb, db, cb, rb):
    wid = lax.axis_index("c") * NS + lax.axis_index("s")
    iota = lax.iota(jnp.int32, 16)

    def chunk_body(ch, carry):
        base = wid * TPW + ch * CH
        pltpu.sync_copy(lg_hbm.at[wid * NCHUNK + ch], inb)

        def tile_body(t, carry2):
            col0 = t * 16
            sl = pl.ds(col0, 16)
            rows = col0 + iota
            gc0 = inb[E + 0, sl]
            gc1 = inb[E + 1, sl]
            gc2 = inb[E + 2, sl]
            gc3 = inb[E + 3, sl]
            cc = inb[E + G, sl]
            gm = gc0
            gidx = jnp.zeros((16,), jnp.int32)
            for g, gc in ((1, gc1), (2, gc2), (3, gc3)):
                gt = gc > gm
                gm = jnp.where(gt, gc, gm)
                gidx = jnp.where(gt, g, gidx)
            gs = (jnp.exp(gc0 - gm) + jnp.exp(gc1 - gm)
                  + jnp.exp(gc2 - gm) + jnp.exp(gc3 - gm))
            hit = [gidx == g for g in range(G)]
            hitf = [jnp.where(h, 1.0, 0.0).astype(jnp.float32) for h in hit]
            ce = []
            for j in range(EPG):
                a0 = inb[j, sl]
                a1 = inb[16 + j, sl]
                a2 = inb[32 + j, sl]
                a3 = inb[48 + j, sl]
                ce.append(jnp.where(hit[0], a0,
                                    jnp.where(hit[1], a1,
                                              jnp.where(hit[2], a2, a3))))
            emax = ce[0]
            for j in range(1, EPG):
                emax = jnp.maximum(emax, ce[j])
            ex = [jnp.exp(v - emax) for v in ce]
            es = ex[0]
            for j in range(1, EPG):
                es = es + ex[j]
            m1 = ex[0]
            i1 = jnp.zeros((16,), jnp.int32)
            m2 = jnp.full((16,), -1.0, jnp.float32)
            i2 = jnp.zeros((16,), jnp.int32)
            for j in range(1, EPG):
                gt1 = ex[j] > m1
                gt2 = jnp.logical_and(ex[j] > m2, jnp.logical_not(gt1))
                m2 = jnp.where(gt1, m1, jnp.where(gt2, ex[j], m2))
                i2 = jnp.where(gt1, i1, jnp.where(gt2, j, i2))
                m1 = jnp.where(gt1, ex[j], m1)
                i1 = jnp.where(gt1, j, i1)
            sel2 = (cc + cc) >= 2.0
            m2k = jnp.where(sel2, m2, 0.0)
            den = jnp.maximum(m1 + m2k, 1e-20)
            n1 = m1 / den
            n2 = m2k / den
            bm = 1.0 / (gs * es)
            one = jnp.float32(1.0)
            zero = jnp.float32(0.0)
            for j in range(EPG):
                eq1 = i1 == j
                eq2 = i2 == j
                dj = (jnp.where(eq1, one, zero)
                      + jnp.where(jnp.logical_and(eq2, sel2), one, zero))
                cj = jnp.where(eq1, n1, zero) + jnp.where(eq2, n2, zero)
                bj = ex[j] * bm
                for g in range(G):
                    cole = jnp.full((16,), g * EPG + j, jnp.int32)
                    hf = hitf[g]
                    plsc.store_scatter(db, [rows, cole], dj * hf)
                    plsc.store_scatter(cb, [rows, cole], cj * hf)
                    plsc.store_scatter(rb, [rows, cole], bj * hf)
            return carry2

        carry = lax.fori_loop(0, NTILE, tile_body, carry)
        pltpu.sync_copy(db, disp_hbm.at[pl.ds(base, CH)])
        pltpu.sync_copy(cb, comb_hbm.at[pl.ds(base, CH)])
        pltpu.sync_copy(rb, rp_hbm.at[pl.ds(base, CH)])
        return carry

    lax.fori_loop(0, NCHUNK, chunk_body, 0)


def _sc_route(logits):
    mesh = plsc.VectorSubcoreMesh(core_axis_name="c", subcore_axis_name="s")
    out_type = (
        jax.ShapeDtypeStruct((T, EP), jnp.float32),
        jax.ShapeDtypeStruct((T, EP), jnp.float32),
        jax.ShapeDtypeStruct((T, EP), jnp.float32),
    )
    scratch = [
        pltpu.VMEM((LROWS, CH), jnp.float32),
        pltpu.VMEM((CH, EP), jnp.float32),
        pltpu.VMEM((CH, EP), jnp.float32),
        pltpu.VMEM((CH, EP), jnp.float32),
    ]
    fn = functools.partial(
        pl.kernel, out_type=out_type, mesh=mesh, scratch_types=scratch,
        compiler_params=pltpu.CompilerParams(needs_layout_passes=False),
    )(_sc_route_body)
    return fn(logits)


RB = 2048


def _tc_aux_body(dref, rref, outref, acc):
    i = pl.program_id(0)

    @pl.when(i == 0)
    def _():
        acc[...] = jnp.zeros_like(acc)

    acc[0:1, :] += jnp.sum(rref[...], axis=0, keepdims=True)
    acc[1:2, :] += jnp.sum(dref[...], axis=0, keepdims=True)

    @pl.when(i == (T // RB) - 1)
    def _():
        a = acc[...]
        outref[0, 0] = jnp.sum(a[0] * a[1]) * (float(E) / (float(T) * float(T)))


def _tc_aux(disp, rp):
    return pl.pallas_call(
        _tc_aux_body,
        grid=(T // RB,),
        in_specs=[
            pl.BlockSpec((RB, E), lambda i: (i, 0)),
            pl.BlockSpec((RB, E), lambda i: (i, 0)),
        ],
        out_specs=pl.BlockSpec(memory_space=pltpu.SMEM),
        out_shape=jax.ShapeDtypeStruct((1, 1), jnp.float32),
        scratch_shapes=[pltpu.VMEM((2, E), jnp.float32)],
    )(disp, rp)


def kernel(x, Wg, We, W1, b1, W2, b2):
    xf = x.reshape(T, D)
    wall = jnp.concatenate(
        [We.reshape(E, D), W1, Wg, jnp.zeros((WPAD - E - H - G, D), jnp.float32)],
        axis=0)
    bias = jnp.zeros((WPAD, 1), jnp.float32).at[E:E + H, 0].set(b1)
    w2pad = jnp.zeros((1, WPAD), jnp.float32).at[0, E:E + H].set(W2[0])
    b2s = b2.reshape(1, 1)

    logits = _tc_logits(xf, wall, bias, w2pad, b2s)
    disp_p, comb_p, rp_p = _sc_route(logits)
    disp = disp_p[:, :E]
    comb = comb_p[:, :E]
    rp = rp_p[:, :E]
    aux = _tc_aux(disp, rp)[0, 0]
    return (disp.reshape(B, S, E), comb.reshape(B, S, E),
            rp.reshape(B, S, E), aux)


# speedup vs baseline: 1.3390x; 1.3390x over previous
import functools

import jax
import jax.numpy as jnp
from jax import lax
from jax.experimental import pallas as pl
from jax.experimental.pallas import tpu as pltpu
from jax.experimental.pallas import tpu_sc as plsc

B, S, D = 4, 8192, 768
G, EPG = 4, 16
E = G * EPG
H = 64
T = B * S
WPAD = 192
LROWS = 128
BT = 512

NC, NS = 2, 16
NW = NC * NS
TPW = T // NW
CH = 128
NCHUNK = TPW // CH
NTILE = CH // 16
EP = 72


def _tc_logits_body(xref, wref, biasref, w2ref, b2ref, outref):
    yt = lax.dot_general(wref[...], xref[...], (((1,), (1,)), ((), ())),
                         preferred_element_type=jnp.float32)
    r = jnp.maximum(yt + biasref[...], 0.0)
    cpre = lax.dot_general(w2ref[...], r, (((1,), (0,)), ((), ())),
                           preferred_element_type=jnp.float32)
    c = jax.nn.sigmoid(cpre + b2ref[0, 0])
    panel = jnp.concatenate(
        [yt[:E], yt[128:132], c, jnp.zeros((LROWS - E - G - 1, BT), jnp.float32)],
        axis=0)
    for cch in range(BT // CH):
        outref[cch] = panel[:, cch * CH:(cch + 1) * CH]


def _tc_logits(xf, wall, bias, w2pad, b2s):
    return pl.pallas_call(
        _tc_logits_body,
        grid=(T // BT,),
        in_specs=[
            pl.BlockSpec((BT, D), lambda i: (i, 0)),
            pl.BlockSpec((WPAD, D), lambda i: (0, 0)),
            pl.BlockSpec((WPAD, 1), lambda i: (0, 0)),
            pl.BlockSpec((1, WPAD), lambda i: (0, 0)),
            pl.BlockSpec(memory_space=pltpu.SMEM),
        ],
        out_specs=pl.BlockSpec((BT // CH, LROWS, CH), lambda i: (i, 0, 0)),
        out_shape=jax.ShapeDtypeStruct((T // CH, LROWS, CH), jnp.float32),
    )(xf, wall, bias, w2pad, b2s)


def _sc_route_body(lg_hbm, disp_hbm, comb_hbm, rp_hbm, inb, db, cb, rb):
    wid = lax.axis_index("c") * NS + lax.axis_index("s")

    def chunk_body(ch, carry):
        base = wid * TPW + ch * CH
        pltpu.sync_copy(lg_hbm.at[wid * NCHUNK + ch], inb)

        def tile_body(t, carry2):
            col0 = t * 16
            sl = pl.ds(col0, 16)
            gc0 = inb[E + 0, sl]
            gc1 = inb[E + 1, sl]
            gc2 = inb[E + 2, sl]
            gc3 = inb[E + 3, sl]
            cc = inb[E + G, sl]
            gm = gc0
            gidx = jnp.zeros((16,), jnp.int32)
            for g, gc in ((1, gc1), (2, gc2), (3, gc3)):
                gt = gc > gm
                gm = jnp.where(gt, gc, gm)
                gidx = jnp.where(gt, g, gidx)
            gs = (jnp.exp(gc0 - gm) + jnp.exp(gc1 - gm)
                  + jnp.exp(gc2 - gm) + jnp.exp(gc3 - gm))
            hit = [gidx == g for g in range(G)]
            hitf = [jnp.where(h, 1.0, 0.0).astype(jnp.float32) for h in hit]
            ce = []
            for j in range(EPG):
                a0 = inb[j, sl]
                a1 = inb[16 + j, sl]
                a2 = inb[32 + j, sl]
                a3 = inb[48 + j, sl]
                ce.append(jnp.where(hit[0], a0,
                                    jnp.where(hit[1], a1,
                                              jnp.where(hit[2], a2, a3))))
            emax = ce[0]
            for j in range(1, EPG):
                emax = jnp.maximum(emax, ce[j])
            ex = [jnp.exp(v - emax) for v in ce]
            es = ex[0]
            for j in range(1, EPG):
                es = es + ex[j]
            m1 = ex[0]
            i1 = jnp.zeros((16,), jnp.int32)
            m2 = jnp.full((16,), -1.0, jnp.float32)
            i2 = jnp.zeros((16,), jnp.int32)
            for j in range(1, EPG):
                gt1 = ex[j] > m1
                gt2 = jnp.logical_and(ex[j] > m2, jnp.logical_not(gt1))
                m2 = jnp.where(gt1, m1, jnp.where(gt2, ex[j], m2))
                i2 = jnp.where(gt1, i1, jnp.where(gt2, j, i2))
                m1 = jnp.where(gt1, ex[j], m1)
                i1 = jnp.where(gt1, j, i1)
            sel2 = (cc + cc) >= 2.0
            m2k = jnp.where(sel2, m2, 0.0)
            den = jnp.maximum(m1 + m2k, 1e-20)
            n1 = m1 / den
            n2 = m2k / den
            bm = 1.0 / (gs * es)
            one = jnp.float32(1.0)
            zero = jnp.float32(0.0)
            for j in range(EPG):
                eq1 = i1 == j
                eq2 = i2 == j
                dj = (jnp.where(eq1, one, zero)
                      + jnp.where(jnp.logical_and(eq2, sel2), one, zero))
                cj = jnp.where(eq1, n1, zero) + jnp.where(eq2, n2, zero)
                bj = ex[j] * bm
                for g in range(G):
                    e = g * EPG + j
                    hf = hitf[g]
                    db[e, sl] = dj * hf
                    cb[e, sl] = cj * hf
                    rb[e, sl] = bj * hf
            return carry2

        carry = lax.fori_loop(0, NTILE, tile_body, carry)
        pltpu.sync_copy(db, disp_hbm.at[wid * NCHUNK + ch])
        pltpu.sync_copy(cb, comb_hbm.at[wid * NCHUNK + ch])
        pltpu.sync_copy(rb, rp_hbm.at[wid * NCHUNK + ch])
        return carry

    lax.fori_loop(0, NCHUNK, chunk_body, 0)


def _sc_route(logits):
    mesh = plsc.VectorSubcoreMesh(core_axis_name="c", subcore_axis_name="s")
    out_type = (
        jax.ShapeDtypeStruct((T // CH, E, CH), jnp.float32),
        jax.ShapeDtypeStruct((T // CH, E, CH), jnp.float32),
        jax.ShapeDtypeStruct((T // CH, E, CH), jnp.float32),
    )
    scratch = [
        pltpu.VMEM((LROWS, CH), jnp.float32),
        pltpu.VMEM((E, CH), jnp.float32),
        pltpu.VMEM((E, CH), jnp.float32),
        pltpu.VMEM((E, CH), jnp.float32),
    ]
    fn = functools.partial(
        pl.kernel, out_type=out_type, mesh=mesh, scratch_types=scratch,
        compiler_params=pltpu.CompilerParams(needs_layout_passes=False),
    )(_sc_route_body)
    return fn(logits)


KB = 8
NFIN = (T // CH) // KB


def _tc_fin_body(dref, cref, rref, do_, co_, ro_, auxref, accd, accr):
    i = pl.program_id(0)

    @pl.when(i == 0)
    def _():
        accd[...] = jnp.zeros_like(accd)
        accr[...] = jnp.zeros_like(accr)

    d3 = dref[...]
    c3 = cref[...]
    r3 = rref[...]
    do_[...] = jnp.transpose(d3, (0, 2, 1)).reshape(KB * CH, E)
    co_[...] = jnp.transpose(c3, (0, 2, 1)).reshape(KB * CH, E)
    ro_[...] = jnp.transpose(r3, (0, 2, 1)).reshape(KB * CH, E)
    accd[...] += jnp.sum(d3, axis=0)
    accr[...] += jnp.sum(r3, axis=0)

    @pl.when(i == NFIN - 1)
    def _():
        rppe = jnp.sum(accr[...], axis=1)
        usage = jnp.sum(accd[...], axis=1)
        auxref[0, 0] = jnp.sum(rppe * usage) * (float(E) / (float(T) * float(T)))


def _tc_fin(disp3, comb3, rp3):
    return pl.pallas_call(
        _tc_fin_body,
        grid=(NFIN,),
        in_specs=[
            pl.BlockSpec((KB, E, CH), lambda i: (i, 0, 0)),
            pl.BlockSpec((KB, E, CH), lambda i: (i, 0, 0)),
            pl.BlockSpec((KB, E, CH), lambda i: (i, 0, 0)),
        ],
        out_specs=[
            pl.BlockSpec((KB * CH, E), lambda i: (i, 0)),
            pl.BlockSpec((KB * CH, E), lambda i: (i, 0)),
            pl.BlockSpec((KB * CH, E), lambda i: (i, 0)),
            pl.BlockSpec(memory_space=pltpu.SMEM),
        ],
        out_shape=[
            jax.ShapeDtypeStruct((T, E), jnp.float32),
            jax.ShapeDtypeStruct((T, E), jnp.float32),
            jax.ShapeDtypeStruct((T, E), jnp.float32),
            jax.ShapeDtypeStruct((1, 1), jnp.float32),
        ],
        scratch_shapes=[pltpu.VMEM((E, CH), jnp.float32),
                        pltpu.VMEM((E, CH), jnp.float32)],
    )(disp3, comb3, rp3)


def kernel(x, Wg, We, W1, b1, W2, b2):
    xf = x.reshape(T, D)
    wall = jnp.concatenate(
        [We.reshape(E, D), W1, Wg, jnp.zeros((WPAD - E - H - G, D), jnp.float32)],
        axis=0)
    bias = jnp.zeros((WPAD, 1), jnp.float32).at[E:E + H, 0].set(b1)
    w2pad = jnp.zeros((1, WPAD), jnp.float32).at[0, E:E + H].set(W2[0])
    b2s = b2.reshape(1, 1)

    logits = _tc_logits(xf, wall, bias, w2pad, b2s)
    disp3, comb3, rp3 = _sc_route(logits)
    disp, comb, rp, aux2 = _tc_fin(disp3, comb3, rp3)
    aux = aux2[0, 0]
    return (disp.reshape(B, S, E), comb.reshape(B, S, E),
            rp.reshape(B, S, E), aux)
